# baseline (speedup 1.0000x reference)
import functools

import jax
import jax.numpy as jnp
from jax import lax
from jax.experimental import pallas as pl
from jax.experimental.pallas import tpu as pltpu
from jax.experimental.pallas import tpu_sc as plsc

K = 3
MEL_MAX = 4096


def _idx_body(t_ref, idx_ref, *, L, M_TILE, zrow):
    b = pl.program_id(0)
    mt = pl.program_id(1)
    t = t_ref[0, 0, :].astype(jnp.float32)
    row = lax.broadcasted_iota(jnp.int32, (L, L), 0)
    col = lax.broadcasted_iota(jnp.int32, (L, L), 1)
    tri = (row <= col).astype(jnp.float32)
    cum = jnp.dot(t.reshape(1, L), tri, preferred_element_type=jnp.float32)
    m = (lax.broadcasted_iota(jnp.int32, (M_TILE, L), 0) + mt * M_TILE).astype(
        jnp.float32
    )
    cnt = jnp.sum((jnp.broadcast_to(cum, (M_TILE, L)) <= m).astype(jnp.float32), axis=1)
    seg = cnt.astype(jnp.int32)
    idx_ref[0, 0, :] = jnp.where(seg < L, b * L + seg, zrow)


def _build_idx(target, B, L, M, zrow):
    M_TILE = 1024
    body = functools.partial(_idx_body, L=L, M_TILE=M_TILE, zrow=zrow)
    idx3 = pl.pallas_call(
        body,
        grid=(B, M // M_TILE),
        in_specs=[pl.BlockSpec((1, 1, L), lambda b, mt: (b, 0, 0))],
        out_specs=pl.BlockSpec((1, 1, M_TILE), lambda b, mt: (b, 0, mt)),
        out_shape=jax.ShapeDtypeStruct((B, 1, M), jnp.int32),
    )(target.reshape(B, 1, L))
    return idx3.reshape(B * M)


def _sc_expand(table, idx, B, M, D):
    N = B * M
    NW = 32
    ROWS_W = N // NW
    CH = 128
    NCH = ROWS_W // CH

    mesh = plsc.VectorSubcoreMesh(core_axis_name="c", subcore_axis_name="s")

    @functools.partial(
        pl.kernel,
        mesh=mesh,
        out_type=jax.ShapeDtypeStruct((N, D), jnp.float32),
        scratch_types=[
            pltpu.VMEM((ROWS_W,), jnp.int32),
            pltpu.VMEM((CH, D), jnp.float32),
            pltpu.VMEM((CH, D), jnp.float32),
            pltpu.SemaphoreType.DMA,
            pltpu.SemaphoreType.DMA,
        ],
    )
    def gather_kernel(table_hbm, idx_hbm, out_hbm, idx_v, buf0, buf1, sem0, sem1):
        wid = lax.axis_index("s") * 2 + lax.axis_index("c")
        base = wid * ROWS_W
        pltpu.sync_copy(idx_hbm.at[pl.ds(base, ROWS_W)], idx_v)
        bufs = (buf0, buf1)
        sems = (sem0, sem1)
        for c in range(NCH):
            buf = bufs[c % 2]
            sem = sems[c % 2]
            pltpu.async_copy(
                table_hbm.at[idx_v.at[pl.ds(c * CH, CH)]], buf, sem
            ).wait()
            pltpu.sync_copy(buf, out_hbm.at[pl.ds(base + c * CH, CH)])

    return gather_kernel(table, idx)


def _dp_body(x_ref, w1_ref, b1_ref, g1_ref, be1_ref, w2_ref, b2_ref, g2_ref,
             be2_ref, wl_ref, bl_ref, dur_ref, *, L, D, FS):
    x = x_ref[0]

    def conv_relu_ln(h, w_ref, b_ref, g_ref, be_ref, C):
        prev = jnp.concatenate([jnp.zeros((1, C), jnp.float32), h[:-1]], axis=0)
        nxt = jnp.concatenate([h[1:], jnp.zeros((1, C), jnp.float32)], axis=0)
        cat = jnp.concatenate([prev, h, nxt], axis=1)
        y = (
            jnp.dot(cat, w_ref[:, :], preferred_element_type=jnp.float32)
            + b_ref[0, :]
        )
        y = jnp.maximum(y, 0.0)
        mu = jnp.mean(y, axis=1, keepdims=True)
        yc = y - mu
        var = jnp.mean(yc * yc, axis=1, keepdims=True)
        return yc * lax.rsqrt(var + 1e-5) * g_ref[0, :] + be_ref[0, :]

    h = conv_relu_ln(x, w1_ref, b1_ref, g1_ref, be1_ref, D)
    h = conv_relu_ln(h, w2_ref, b2_ref, g2_ref, be2_ref, FS)
    dur_ref[0, 0, :] = jnp.sum(h * wl_ref[0, :], axis=1) + bl_ref[0, :]


def _duration_predictor(x, W1, b1, g1, be1, W2, b2, g2, be2, Wl, bl, B, L, D, FS):
    W1r = jnp.transpose(W1, (2, 1, 0)).reshape(K * D, FS)
    W2r = jnp.transpose(W2, (2, 1, 0)).reshape(K * FS, FS)
    body = functools.partial(_dp_body, L=L, D=D, FS=FS)
    vec = lambda n: pl.BlockSpec((1, n), lambda b: (0, 0))
    dur3 = pl.pallas_call(
        body,
        grid=(B,),
        in_specs=[
            pl.BlockSpec((1, L, D), lambda b: (b, 0, 0)),
            pl.BlockSpec((K * D, FS), lambda b: (0, 0)),
            vec(FS),
            vec(FS),
            vec(FS),
            pl.BlockSpec((K * FS, FS), lambda b: (0, 0)),
            vec(FS),
            vec(FS),
            vec(FS),
            vec(FS),
            vec(1),
        ],
        out_specs=pl.BlockSpec((1, 1, L), lambda b: (b, 0, 0)),
        out_shape=jax.ShapeDtypeStruct((B, 1, L), jnp.float32),
    )(
        x,
        W1r,
        b1.reshape(1, FS),
        g1.reshape(1, FS),
        be1.reshape(1, FS),
        W2r,
        b2.reshape(1, FS),
        g2.reshape(1, FS),
        be2.reshape(1, FS),
        Wl.reshape(1, FS),
        bl.reshape(1, 1),
    )
    return dur3.reshape(B, L)


def kernel(x, target, mel_max_length, W1, b1, g1, be1, W2, b2, g2, be2, Wl, bl):
    del mel_max_length
    B, L, D = x.shape
    FS = W1.shape[0]
    M = MEL_MAX
    zrow = B * L

    idx = _build_idx(target.astype(jnp.int32), B, L, M, zrow)
    table = jnp.concatenate([x.reshape(B * L, D), jnp.zeros((8, D), x.dtype)], axis=0)
    out = _sc_expand(table, idx, B, M, D).reshape(B, M, D)
    dur = _duration_predictor(x, W1, b1, g1, be1, W2, b2, g2, be2, Wl, bl, B, L, D, FS)
    return (out, dur)


# speedup vs baseline: 1.0019x; 1.0019x over previous
import functools

import jax
import jax.numpy as jnp
from jax import lax
from jax.experimental import pallas as pl
from jax.experimental.pallas import tpu as pltpu
from jax.experimental.pallas import tpu_sc as plsc

K = 3
MEL_MAX = 4096


def _idx_body(t_ref, idx_ref, *, L, M_TILE, zrow):
    b = pl.program_id(0)
    mt = pl.program_id(1)
    t = t_ref[0, 0, :].astype(jnp.float32)
    row = lax.broadcasted_iota(jnp.int32, (L, L), 0)
    col = lax.broadcasted_iota(jnp.int32, (L, L), 1)
    tri = (row <= col).astype(jnp.float32)
    cum = jnp.dot(t.reshape(1, L), tri, preferred_element_type=jnp.float32)
    m = (lax.broadcasted_iota(jnp.int32, (M_TILE, L), 0) + mt * M_TILE).astype(
        jnp.float32
    )
    cnt = jnp.sum((jnp.broadcast_to(cum, (M_TILE, L)) <= m).astype(jnp.float32), axis=1)
    seg = cnt.astype(jnp.int32)
    idx_ref[0, 0, :] = jnp.where(seg < L, b * L + seg, zrow)


def _build_idx(target, B, L, M, zrow):
    M_TILE = 1024
    body = functools.partial(_idx_body, L=L, M_TILE=M_TILE, zrow=zrow)
    idx3 = pl.pallas_call(
        body,
        grid=(B, M // M_TILE),
        in_specs=[pl.BlockSpec((1, 1, L), lambda b, mt: (b, 0, 0))],
        out_specs=pl.BlockSpec((1, 1, M_TILE), lambda b, mt: (b, 0, mt)),
        out_shape=jax.ShapeDtypeStruct((B, 1, M), jnp.int32),
    )(target.reshape(B, 1, L))
    return idx3.reshape(B * M)


def _sc_expand(table, idx, B, M, D):
    N = B * M
    NW = 32
    ROWS_W = N // NW
    CH = 128
    NCH = ROWS_W // CH

    NBUF = 3
    mesh = plsc.VectorSubcoreMesh(core_axis_name="c", subcore_axis_name="s")

    @functools.partial(
        pl.kernel,
        mesh=mesh,
        out_type=jax.ShapeDtypeStruct((N, D), jnp.float32),
        scratch_types=[
            [pltpu.VMEM((CH,), jnp.int32) for _ in range(NBUF)],
            [pltpu.VMEM((CH, D), jnp.float32) for _ in range(NBUF)],
            [pltpu.SemaphoreType.DMA for _ in range(NBUF)],
            [pltpu.SemaphoreType.DMA for _ in range(NBUF)],
        ],
    )
    def gather_kernel(table_hbm, idx_hbm, out_hbm, idxb, bufs, gsems, wsems):
        wid = lax.axis_index("s") * 2 + lax.axis_index("c")
        base = wid * ROWS_W

        def fire(c):
            s = c % NBUF
            pltpu.sync_copy(idx_hbm.at[wid].at[c], idxb[s])
            return pltpu.async_copy(table_hbm.at[idxb[s]], bufs[s], gsems[s])

        g = [None] * NBUF
        w = [None] * NBUF
        for c in range(min(NBUF - 1, NCH)):
            g[c] = fire(c)
        for c in range(NCH):
            f = c + NBUF - 1
            if f < NCH:
                if c >= 1:
                    w[(c - 1) % NBUF].wait()
                g[f % NBUF] = fire(f)
            g[c % NBUF].wait()
            w[c % NBUF] = pltpu.async_copy(
                bufs[c % NBUF], out_hbm.at[pl.ds(base + c * CH, CH)], wsems[c % NBUF]
            )
        for c in range(max(0, NCH - NBUF), NCH):
            w[c % NBUF].wait()

    return gather_kernel(table, idx.reshape(NW, NCH, CH))


def _dp_body(x_ref, w1_ref, b1_ref, g1_ref, be1_ref, w2_ref, b2_ref, g2_ref,
             be2_ref, wl_ref, bl_ref, dur_ref, *, L, D, FS):
    x = x_ref[0]

    def conv_relu_ln(h, w_ref, b_ref, g_ref, be_ref, C):
        prev = jnp.concatenate([jnp.zeros((1, C), jnp.float32), h[:-1]], axis=0)
        nxt = jnp.concatenate([h[1:], jnp.zeros((1, C), jnp.float32)], axis=0)
        cat = jnp.concatenate([prev, h, nxt], axis=1)
        y = (
            jnp.dot(cat, w_ref[:, :], preferred_element_type=jnp.float32)
            + b_ref[0, :]
        )
        y = jnp.maximum(y, 0.0)
        mu = jnp.mean(y, axis=1, keepdims=True)
        yc = y - mu
        var = jnp.mean(yc * yc, axis=1, keepdims=True)
        return yc * lax.rsqrt(var + 1e-5) * g_ref[0, :] + be_ref[0, :]

    h = conv_relu_ln(x, w1_ref, b1_ref, g1_ref, be1_ref, D)
    h = conv_relu_ln(h, w2_ref, b2_ref, g2_ref, be2_ref, FS)
    dur_ref[0, 0, :] = jnp.sum(h * wl_ref[0, :], axis=1) + bl_ref[0, :]


def _duration_predictor(x, W1, b1, g1, be1, W2, b2, g2, be2, Wl, bl, B, L, D, FS):
    W1r = jnp.transpose(W1, (2, 1, 0)).reshape(K * D, FS)
    W2r = jnp.transpose(W2, (2, 1, 0)).reshape(K * FS, FS)
    body = functools.partial(_dp_body, L=L, D=D, FS=FS)
    vec = lambda n: pl.BlockSpec((1, n), lambda b: (0, 0))
    dur3 = pl.pallas_call(
        body,
        grid=(B,),
        in_specs=[
            pl.BlockSpec((1, L, D), lambda b: (b, 0, 0)),
            pl.BlockSpec((K * D, FS), lambda b: (0, 0)),
            vec(FS),
            vec(FS),
            vec(FS),
            pl.BlockSpec((K * FS, FS), lambda b: (0, 0)),
            vec(FS),
            vec(FS),
            vec(FS),
            vec(FS),
            vec(1),
        ],
        out_specs=pl.BlockSpec((1, 1, L), lambda b: (b, 0, 0)),
        out_shape=jax.ShapeDtypeStruct((B, 1, L), jnp.float32),
    )(
        x,
        W1r,
        b1.reshape(1, FS),
        g1.reshape(1, FS),
        be1.reshape(1, FS),
        W2r,
        b2.reshape(1, FS),
        g2.reshape(1, FS),
        be2.reshape(1, FS),
        Wl.reshape(1, FS),
        bl.reshape(1, 1),
    )
    return dur3.reshape(B, L)


def kernel(x, target, mel_max_length, W1, b1, g1, be1, W2, b2, g2, be2, Wl, bl):
    del mel_max_length
    B, L, D = x.shape
    FS = W1.shape[0]
    M = MEL_MAX
    zrow = B * L

    idx = _build_idx(target.astype(jnp.int32), B, L, M, zrow)
    table = jnp.concatenate([x.reshape(B * L, D), jnp.zeros((8, D), x.dtype)], axis=0)
    out = _sc_expand(table, idx, B, M, D).reshape(B, M, D)
    dur = _duration_predictor(x, W1, b1, g1, be1, W2, b2, g2, be2, Wl, bl, B, L, D, FS)
    return (out, dur)


# speedup vs baseline: 15.7332x; 15.7039x over previous
import functools

import jax
import jax.numpy as jnp
from jax import lax
from jax.experimental import pallas as pl
from jax.experimental.pallas import tpu as pltpu
from jax.experimental.pallas import tpu_sc as plsc

K = 3
MEL_MAX = 4096


def _expand_body(t_ref, x_ref, out_ref, *, L, D, MT):
    mt = pl.program_id(1)
    t = t_ref[0, 0, :].astype(jnp.float32)
    row = lax.broadcasted_iota(jnp.int32, (L, L), 0)
    col = lax.broadcasted_iota(jnp.int32, (L, L), 1)
    tri = (row <= col).astype(jnp.float32)
    cum = jnp.dot(t.reshape(1, L), tri, preferred_element_type=jnp.float32)
    starts = cum - t.reshape(1, L)
    m = (lax.broadcasted_iota(jnp.int32, (MT, L), 0) + mt * MT).astype(jnp.float32)
    ge = jnp.broadcast_to(starts, (MT, L)) <= m
    lt = m < jnp.broadcast_to(cum, (MT, L))
    align = (ge & lt).astype(jnp.bfloat16)
    xb = x_ref[0].astype(jnp.bfloat16)
    out_ref[0] = jnp.dot(align, xb, preferred_element_type=jnp.float32)


def _expand(x, target, B, L, D, M):
    MT = 1024
    body = functools.partial(_expand_body, L=L, D=D, MT=MT)
    return pl.pallas_call(
        body,
        grid=(B, M // MT),
        in_specs=[
            pl.BlockSpec((1, 1, L), lambda b, mt: (b, 0, 0)),
            pl.BlockSpec((1, L, D), lambda b, mt: (b, 0, 0)),
        ],
        out_specs=pl.BlockSpec((1, MT, D), lambda b, mt: (b, mt, 0)),
        out_shape=jax.ShapeDtypeStruct((B, M, D), jnp.float32),
    )(target.reshape(B, 1, L), x)


def _dp_body(x_ref, w1_ref, b1_ref, g1_ref, be1_ref, w2_ref, b2_ref, g2_ref,
             be2_ref, wl_ref, bl_ref, dur_ref, *, L, D, FS):
    x = x_ref[0]

    def conv_relu_ln(h, w_ref, b_ref, g_ref, be_ref, C):
        prev = jnp.concatenate([jnp.zeros((1, C), jnp.float32), h[:-1]], axis=0)
        nxt = jnp.concatenate([h[1:], jnp.zeros((1, C), jnp.float32)], axis=0)
        cat = jnp.concatenate([prev, h, nxt], axis=1)
        y = (
            jnp.dot(cat, w_ref[:, :], preferred_element_type=jnp.float32)
            + b_ref[0, :]
        )
        y = jnp.maximum(y, 0.0)
        mu = jnp.mean(y, axis=1, keepdims=True)
        yc = y - mu
        var = jnp.mean(yc * yc, axis=1, keepdims=True)
        return yc * lax.rsqrt(var + 1e-5) * g_ref[0, :] + be_ref[0, :]

    h = conv_relu_ln(x, w1_ref, b1_ref, g1_ref, be1_ref, D)
    h = conv_relu_ln(h, w2_ref, b2_ref, g2_ref, be2_ref, FS)
    dur_ref[0, 0, :] = jnp.sum(h * wl_ref[0, :], axis=1) + bl_ref[0, :]


def _duration_predictor(x, W1, b1, g1, be1, W2, b2, g2, be2, Wl, bl, B, L, D, FS):
    W1r = jnp.transpose(W1, (2, 1, 0)).reshape(K * D, FS)
    W2r = jnp.transpose(W2, (2, 1, 0)).reshape(K * FS, FS)
    body = functools.partial(_dp_body, L=L, D=D, FS=FS)
    vec = lambda n: pl.BlockSpec((1, n), lambda b: (0, 0))
    dur3 = pl.pallas_call(
        body,
        grid=(B,),
        in_specs=[
            pl.BlockSpec((1, L, D), lambda b: (b, 0, 0)),
            pl.BlockSpec((K * D, FS), lambda b: (0, 0)),
            vec(FS),
            vec(FS),
            vec(FS),
            pl.BlockSpec((K * FS, FS), lambda b: (0, 0)),
            vec(FS),
            vec(FS),
            vec(FS),
            vec(FS),
            vec(1),
        ],
        out_specs=pl.BlockSpec((1, 1, L), lambda b: (b, 0, 0)),
        out_shape=jax.ShapeDtypeStruct((B, 1, L), jnp.float32),
    )(
        x,
        W1r,
        b1.reshape(1, FS),
        g1.reshape(1, FS),
        be1.reshape(1, FS),
        W2r,
        b2.reshape(1, FS),
        g2.reshape(1, FS),
        be2.reshape(1, FS),
        Wl.reshape(1, FS),
        bl.reshape(1, 1),
    )
    return dur3.reshape(B, L)


def kernel(x, target, mel_max_length, W1, b1, g1, be1, W2, b2, g2, be2, Wl, bl):
    del mel_max_length
    B, L, D = x.shape
    FS = W1.shape[0]
    M = MEL_MAX

    out = _expand(x, target.astype(jnp.int32), B, L, D, M)
    dur = _duration_predictor(x, W1, b1, g1, be1, W2, b2, g2, be2, Wl, bl, B, L, D, FS)
    return (out, dur)


# speedup vs baseline: 26.7928x; 1.7030x over previous
import functools

import jax
import jax.numpy as jnp
from jax import lax
from jax.experimental import pallas as pl

K = 3
MEL_MAX = 4096


def _fused_body(t_ref, x_ref, w1_ref, b1_ref, g1_ref, be1_ref, w2_ref, b2_ref,
                g2_ref, be2_ref, wl_ref, bl_ref, out_ref, dur_ref, *, L, D, FS, M):
    x = x_ref[0]

    t = t_ref[0, 0, :].astype(jnp.float32)
    row = lax.broadcasted_iota(jnp.int32, (L, L), 0)
    col = lax.broadcasted_iota(jnp.int32, (L, L), 1)
    tri = (row <= col).astype(jnp.float32)
    cum = jnp.dot(t.reshape(1, L), tri, preferred_element_type=jnp.float32)
    starts = cum - t.reshape(1, L)
    m = lax.broadcasted_iota(jnp.int32, (M, L), 0).astype(jnp.float32)
    align = (
        (jnp.broadcast_to(starts, (M, L)) <= m) & (m < jnp.broadcast_to(cum, (M, L)))
    ).astype(jnp.bfloat16)
    xb = x.astype(jnp.bfloat16)
    out_ref[0] = jnp.dot(align, xb, preferred_element_type=jnp.float32)

    def conv_relu_ln(h, w_ref, b_ref, g_ref, be_ref, C):
        prev = jnp.concatenate([jnp.zeros((1, C), jnp.float32), h[:-1]], axis=0)
        nxt = jnp.concatenate([h[1:], jnp.zeros((1, C), jnp.float32)], axis=0)
        cat = jnp.concatenate([prev, h, nxt], axis=1)
        y = jnp.dot(cat, w_ref[:, :], preferred_element_type=jnp.float32) + b_ref[0, :]
        y = jnp.maximum(y, 0.0)
        mu = jnp.mean(y, axis=1, keepdims=True)
        yc = y - mu
        var = jnp.mean(yc * yc, axis=1, keepdims=True)
        return yc * lax.rsqrt(var + 1e-5) * g_ref[0, :] + be_ref[0, :]

    h = conv_relu_ln(x, w1_ref, b1_ref, g1_ref, be1_ref, D)
    h = conv_relu_ln(h, w2_ref, b2_ref, g2_ref, be2_ref, FS)
    dur_ref[0, 0, :] = jnp.sum(h * wl_ref[0, :], axis=1) + bl_ref[0, :]


def kernel(x, target, mel_max_length, W1, b1, g1, be1, W2, b2, g2, be2, Wl, bl):
    del mel_max_length
    B, L, D = x.shape
    FS = W1.shape[0]
    M = MEL_MAX

    W1r = jnp.transpose(W1, (2, 1, 0)).reshape(K * D, FS)
    W2r = jnp.transpose(W2, (2, 1, 0)).reshape(K * FS, FS)

    body = functools.partial(_fused_body, L=L, D=D, FS=FS, M=M)
    vec = lambda n: pl.BlockSpec((1, n), lambda b: (0, 0))
    out, dur3 = pl.pallas_call(
        body,
        grid=(B,),
        in_specs=[
            pl.BlockSpec((1, 1, L), lambda b: (b, 0, 0)),
            pl.BlockSpec((1, L, D), lambda b: (b, 0, 0)),
            pl.BlockSpec((K * D, FS), lambda b: (0, 0)),
            vec(FS),
            vec(FS),
            vec(FS),
            pl.BlockSpec((K * FS, FS), lambda b: (0, 0)),
            vec(FS),
            vec(FS),
            vec(FS),
            vec(FS),
            vec(1),
        ],
        out_specs=[
            pl.BlockSpec((1, M, D), lambda b: (b, 0, 0)),
            pl.BlockSpec((1, 1, L), lambda b: (b, 0, 0)),
        ],
        out_shape=[
            jax.ShapeDtypeStruct((B, M, D), jnp.float32),
            jax.ShapeDtypeStruct((B, 1, L), jnp.float32),
        ],
    )(
        target.astype(jnp.int32).reshape(B, 1, L),
        x,
        W1r,
        b1.reshape(1, FS),
        g1.reshape(1, FS),
        be1.reshape(1, FS),
        W2r,
        b2.reshape(1, FS),
        g2.reshape(1, FS),
        be2.reshape(1, FS),
        Wl.reshape(1, FS),
        bl.reshape(1, 1),
    )
    return (out, dur3.reshape(B, L))
